# baseline (speedup 1.0000x reference)
import functools

import jax
import jax.numpy as jnp
from jax import lax
from jax.experimental import pallas as pl
from jax.experimental.pallas import tpu as pltpu
from jax.experimental.pallas import tpu_sc as plsc

NUM_EMB = 1_000_000
DIM = 16
BATCH = 16384
HIST = 20

NUM_CORES = 2
NUM_SUBCORES = 16
D_PER_CORE = DIM // NUM_CORES
B_BLOCK = BATCH // NUM_SUBCORES
STAGE_CHUNK = 124928
STAGE_TAIL_OFF = 8 * STAGE_CHUNK
STAGE_TAIL = NUM_EMB - STAGE_TAIL_OFF

_mesh = plsc.VectorSubcoreMesh(core_axis_name="c", subcore_axis_name="s")


@functools.partial(
    pl.kernel,
    mesh=_mesh,
    out_type=jax.ShapeDtypeStruct((HIST, DIM, BATCH), jnp.float32),
    scratch_types=[
        pltpu.VMEM((B_BLOCK,), jnp.int32),
        pltpu.VMEM((B_BLOCK,), jnp.int32),
        pltpu.VMEM((1, B_BLOCK), jnp.float32),
        pltpu.VMEM((1, B_BLOCK), jnp.float32),
        pltpu.VMEM((1, B_BLOCK), jnp.float32),
        pltpu.VMEM_SHARED((1, NUM_EMB), jnp.float32),
        pltpu.VMEM_SHARED((1, NUM_EMB), jnp.float32),
        pltpu.SemaphoreType.DMA,
        pltpu.SemaphoreType.DMA,
        pltpu.SemaphoreType.DMA,
        pltpu.SemaphoreType.DMA,
        pltpu.SemaphoreType.DMA,
        pltpu.SemaphoreType.DMA,
        pltpu.SemaphoreType.DMA,
        pltpu.SemaphoreType.DMA,
        pltpu.SemaphoreType.DMA,
    ],
)
def _emb_lookup(
    xt_hbm, tt_hbm, out_hbm,
    idx0, idx1, rows0, rows1, rows2, drow0, drow1,
    ssem, isem0, isem1, gsem0, gsem1, gsem2, wsem0, wsem1, wsem2,
):
    cid = lax.axis_index("c")
    sid = lax.axis_index("s")
    dbase = cid * D_PER_CORE
    b0 = sid * B_BLOCK
    idxs = [idx0, idx1]
    isem = [isem0, isem1]
    rows = [rows0, rows1, rows2]
    gsem = [gsem0, gsem1, gsem2]
    wsem = [wsem0, wsem1, wsem2]
    drow = [drow0, drow1]
    T = D_PER_CORE * HIST

    def idx_start(t):
        h = t % HIST
        pltpu.async_copy(
            xt_hbm.at[h, pl.ds(b0, B_BLOCK)], idxs[t % 2], isem[t % 2]
        )

    def idx_wait(t):
        h = t % HIST
        pltpu.make_async_copy(
            xt_hbm.at[h, pl.ds(b0, B_BLOCK)], idxs[t % 2], isem[t % 2]
        ).wait()

    def _stage_args(d):
        grp = (d % 2) * 8
        i = sid - grp
        off = pl.multiple_of(i * STAGE_CHUNK, 128)
        src = tt_hbm.at[pl.ds(dbase + d, 1), pl.ds(off, STAGE_CHUNK)]
        dst = drow[d % 2].at[:, pl.ds(off, STAGE_CHUNK)]
        tsrc = tt_hbm.at[pl.ds(dbase + d, 1), pl.ds(STAGE_TAIL_OFF, STAGE_TAIL)]
        tdst = drow[d % 2].at[:, pl.ds(STAGE_TAIL_OFF, STAGE_TAIL)]
        return grp, src, dst, tsrc, tdst

    def stage_start(d):
        grp, src, dst, tsrc, tdst = _stage_args(d)

        @pl.when((sid >= grp) & (sid < grp + 8))
        def _():
            pltpu.async_copy(src, dst, ssem)

        @pl.when(sid == grp)
        def _():
            pltpu.async_copy(tsrc, tdst, ssem)

    def stage_wait(d):
        grp, src, dst, tsrc, tdst = _stage_args(d)

        @pl.when((sid >= grp) & (sid < grp + 8))
        def _():
            pltpu.make_async_copy(src, dst, ssem).wait()

        @pl.when(sid == grp)
        def _():
            pltpu.make_async_copy(tsrc, tdst, ssem).wait()

    stage_start(0)
    idx_start(0)
    idx_start(1)
    stage_wait(0)
    plsc.subcore_barrier()

    writes = [None, None, None]
    gathers = [None, None, None]
    for d in range(D_PER_CORE):
        buf = d % 2
        if d + 1 < D_PER_CORE:
            stage_start(d + 1)

        for h in range(HIST):
            t = d * HIST + h
            rb = t % 3
            idx_wait(t)
            if writes[rb] is not None:
                writes[rb].wait()
                writes[rb] = None
            gathers[rb] = pltpu.async_copy(
                drow[buf].at[0].at[idxs[t % 2]],
                rows[rb].at[0],
                gsem[rb],
            )
            if h >= 1:
                pb = (t - 1) % 3
                gathers[pb].wait()
                writes[pb] = pltpu.async_copy(
                    rows[pb],
                    out_hbm.at[h - 1, pl.ds(dbase + d, 1), pl.ds(b0, B_BLOCK)],
                    wsem[pb],
                )
            if 1 <= t < T - 1:
                idx_start(t + 1)
        tl = (d * HIST + HIST - 1) % 3
        gathers[tl].wait()
        writes[tl] = pltpu.async_copy(
            rows[tl],
            out_hbm.at[HIST - 1, pl.ds(dbase + d, 1), pl.ds(b0, B_BLOCK)],
            wsem[tl],
        )

        if d + 1 < D_PER_CORE:
            stage_wait(d + 1)
        plsc.subcore_barrier()

    for rb in range(3):
        if writes[rb] is not None:
            writes[rb].wait()


def kernel(x, emb_weight):
    out = _emb_lookup(x.T.astype(jnp.int32), emb_weight.T)
    return out.transpose(2, 0, 1)


# speedup vs baseline: 1.3426x; 1.3426x over previous
import functools

import jax
import jax.numpy as jnp
from jax import lax
from jax.experimental import pallas as pl
from jax.experimental.pallas import tpu as pltpu
from jax.experimental.pallas import tpu_sc as plsc

NUM_EMB = 1_000_000
DIM = 16
BATCH = 16384
HIST = 20

NUM_CORES = 2
NUM_SUBCORES = 16
D_PER_CORE = DIM // NUM_CORES
B_BLOCK = BATCH // NUM_SUBCORES
STAGE_CHUNK = 124928
STAGE_TAIL_OFF = 8 * STAGE_CHUNK
STAGE_TAIL = NUM_EMB - STAGE_TAIL_OFF
PRIME_CHUNK = 62464

_mesh = plsc.VectorSubcoreMesh(core_axis_name="c", subcore_axis_name="s")


@functools.partial(
    pl.kernel,
    mesh=_mesh,
    out_type=jax.ShapeDtypeStruct((HIST, DIM, BATCH), jnp.float32),
    scratch_types=[
        pltpu.VMEM((B_BLOCK,), jnp.int32),
        pltpu.VMEM((B_BLOCK,), jnp.int32),
        pltpu.VMEM((B_BLOCK,), jnp.int32),
        pltpu.VMEM((1, B_BLOCK), jnp.float32),
        pltpu.VMEM((1, B_BLOCK), jnp.float32),
        pltpu.VMEM_SHARED((1, NUM_EMB), jnp.float32),
        pltpu.VMEM_SHARED((1, NUM_EMB), jnp.float32),
        pltpu.SemaphoreType.DMA,
        pltpu.SemaphoreType.DMA,
        pltpu.SemaphoreType.DMA,
        pltpu.SemaphoreType.DMA,
        pltpu.SemaphoreType.DMA,
        pltpu.SemaphoreType.DMA,
        pltpu.SemaphoreType.DMA,
        pltpu.SemaphoreType.DMA,
    ],
)
def _emb_lookup(
    xt_hbm, tt_hbm, out_hbm,
    idx0, idx1, idx2, rows0, rows1, drow0, drow1,
    ssem, isem0, isem1, isem2, gsem0, gsem1, wsem0, wsem1,
):
    cid = lax.axis_index("c")
    sid = lax.axis_index("s")
    dbase = cid * D_PER_CORE
    b0 = sid * B_BLOCK
    idxs = [idx0, idx1, idx2]
    isem = [isem0, isem1, isem2]
    rows = [rows0, rows1]
    gsem = [gsem0, gsem1]
    wsem = [wsem0, wsem1]
    drow = [drow0, drow1]
    T = D_PER_CORE * HIST

    def idx_start(t):
        h = t % HIST
        pltpu.async_copy(
            xt_hbm.at[h, pl.ds(b0, B_BLOCK)], idxs[t % 3], isem[t % 3]
        )

    def idx_wait(t):
        h = t % HIST
        pltpu.make_async_copy(
            xt_hbm.at[h, pl.ds(b0, B_BLOCK)], idxs[t % 3], isem[t % 3]
        ).wait()

    def _stage_args(d):
        grp = (d % 2) * 8
        i = sid - grp
        off = pl.multiple_of(i * STAGE_CHUNK, 128)
        src = tt_hbm.at[pl.ds(dbase + d, 1), pl.ds(off, STAGE_CHUNK)]
        dst = drow[d % 2].at[:, pl.ds(off, STAGE_CHUNK)]
        tsrc = tt_hbm.at[pl.ds(dbase + d, 1), pl.ds(STAGE_TAIL_OFF, STAGE_TAIL)]
        tdst = drow[d % 2].at[:, pl.ds(STAGE_TAIL_OFF, STAGE_TAIL)]
        return grp, src, dst, tsrc, tdst

    def stage_start(d):
        grp, src, dst, tsrc, tdst = _stage_args(d)

        @pl.when((sid >= grp) & (sid < grp + 8))
        def _():
            pltpu.async_copy(src, dst, ssem)

        @pl.when(sid == grp)
        def _():
            pltpu.async_copy(tsrc, tdst, ssem)

    def stage_wait(d):
        grp, src, dst, tsrc, tdst = _stage_args(d)

        @pl.when((sid >= grp) & (sid < grp + 8))
        def _():
            pltpu.make_async_copy(src, dst, ssem).wait()

        @pl.when(sid == grp)
        def _():
            pltpu.make_async_copy(tsrc, tdst, ssem).wait()

    poff = pl.multiple_of(sid * PRIME_CHUNK, 128)
    psrc = tt_hbm.at[pl.ds(dbase, 1), pl.ds(poff, PRIME_CHUNK)]
    pdst = drow[0].at[:, pl.ds(poff, PRIME_CHUNK)]
    ptsrc = tt_hbm.at[pl.ds(dbase, 1), pl.ds(STAGE_TAIL_OFF, STAGE_TAIL)]
    ptdst = drow[0].at[:, pl.ds(STAGE_TAIL_OFF, STAGE_TAIL)]
    pltpu.async_copy(psrc, pdst, ssem)

    @pl.when(sid == 0)
    def _():
        pltpu.async_copy(ptsrc, ptdst, ssem)

    idx_start(0)
    idx_start(1)
    idx_start(2)
    pltpu.make_async_copy(psrc, pdst, ssem).wait()

    @pl.when(sid == 0)
    def _():
        pltpu.make_async_copy(ptsrc, ptdst, ssem).wait()

    plsc.subcore_barrier()

    writes = [None, None]
    gathers = [None, None]
    for d in range(D_PER_CORE):
        buf = d % 2
        if d + 1 < D_PER_CORE:
            stage_start(d + 1)

        for h in range(HIST):
            t = d * HIST + h
            rb = t % 2
            idx_wait(t)
            if writes[rb] is not None:
                writes[rb].wait()
                writes[rb] = None
            gathers[rb] = pltpu.async_copy(
                drow[buf].at[0].at[idxs[t % 3]],
                rows[rb].at[0],
                gsem[rb],
            )
            if h >= 1:
                pb = 1 - rb
                gathers[pb].wait()
                writes[pb] = pltpu.async_copy(
                    rows[pb],
                    out_hbm.at[h - 1, pl.ds(dbase + d, 1), pl.ds(b0, B_BLOCK)],
                    wsem[pb],
                )
            if 1 <= t < T - 2:
                idx_start(t + 2)
        tl = (HIST - 1) % 2
        gathers[tl].wait()
        writes[tl] = pltpu.async_copy(
            rows[tl],
            out_hbm.at[HIST - 1, pl.ds(dbase + d, 1), pl.ds(b0, B_BLOCK)],
            wsem[tl],
        )

        if d + 1 < D_PER_CORE:
            stage_wait(d + 1)
        plsc.subcore_barrier()

    for rb in range(2):
        if writes[rb] is not None:
            writes[rb].wait()


def kernel(x, emb_weight):
    out = _emb_lookup(x.T.astype(jnp.int32), emb_weight.T)
    return out.transpose(2, 0, 1)
